# baseline (speedup 1.0000x reference)
import jax
import jax.numpy as jnp
from jax import lax
from jax.experimental import pallas as pl
from jax.experimental.pallas import tpu as pltpu
from jax.experimental.pallas import tpu_sc as plsc

_BATCH = 16384
_NC = 2
_NS = 16
_L = 16
_NW = _NC * _NS
_CHUNK = _BATCH // _NW

_STUDENT_NUM = 1000000
_EXER_N = 100000
_OFF_A = _STUDENT_NUM
_OFF_B = _STUDENT_NUM + _EXER_N
_OFF_C = _STUDENT_NUM + 2 * _EXER_N

_VALUE_RANGE = 8.0
_A_RANGE = 3.0


def _sigmoid(x):
    return 1.0 / (1.0 + jnp.exp(-x))


def _body(stu_hbm, exer_hbm, tab_hbm, out_hbm,
          idx_s, idx_e, idx_a, idx_b, idx_c,
          th_v, a_v, b_v, c_v, out_v, sem):
    wid = lax.axis_index("s") * _NC + lax.axis_index("c")
    base = wid * _CHUNK
    pltpu.sync_copy(stu_hbm.at[pl.ds(base, _CHUNK)], idx_s)
    pltpu.sync_copy(exer_hbm.at[pl.ds(base, _CHUNK)], idx_e)
    for i in range(_CHUNK // _L):
        sl = pl.ds(i * _L, _L)
        e = idx_e[sl]
        idx_a[sl] = e + _OFF_A
        idx_b[sl] = e + _OFF_B
        idx_c[sl] = e + _OFF_C
    cp1 = pltpu.async_copy(tab_hbm.at[idx_s], th_v, sem)
    cp2 = pltpu.async_copy(tab_hbm.at[idx_a], a_v, sem)
    cp3 = pltpu.async_copy(tab_hbm.at[idx_b], b_v, sem)
    cp4 = pltpu.async_copy(tab_hbm.at[idx_c], c_v, sem)
    cp1.wait()
    cp2.wait()
    cp3.wait()
    cp4.wait()
    for i in range(_CHUNK // _L):
        sl = pl.ds(i * _L, _L)
        th = _VALUE_RANGE * (_sigmoid(th_v[sl]) - 0.5)
        bb = _VALUE_RANGE * (_sigmoid(b_v[sl]) - 0.5)
        aa = _A_RANGE * _sigmoid(a_v[sl])
        cc = _sigmoid(c_v[sl])
        out_v[sl] = cc + (1.0 - cc) / (1.0 + jnp.exp(-1.702 * aa * (th - bb)))
    pltpu.sync_copy(out_v, out_hbm.at[pl.ds(base, _CHUNK)])


def kernel(stu_id, input_exercise, theta_w, a_w, b_w, c_w):
    table = jnp.concatenate(
        [theta_w, a_w, b_w, c_w], axis=0).reshape(-1)
    mesh = plsc.VectorSubcoreMesh(
        core_axis_name="c", subcore_axis_name="s",
        num_cores=_NC, num_subcores=_NS)
    run = pl.kernel(
        _body,
        out_type=jax.ShapeDtypeStruct((_BATCH,), jnp.float32),
        mesh=mesh,
        scratch_types=[
            pltpu.VMEM((_CHUNK,), jnp.int32),
            pltpu.VMEM((_CHUNK,), jnp.int32),
            pltpu.VMEM((_CHUNK,), jnp.int32),
            pltpu.VMEM((_CHUNK,), jnp.int32),
            pltpu.VMEM((_CHUNK,), jnp.int32),
            pltpu.VMEM((_CHUNK,), jnp.float32),
            pltpu.VMEM((_CHUNK,), jnp.float32),
            pltpu.VMEM((_CHUNK,), jnp.float32),
            pltpu.VMEM((_CHUNK,), jnp.float32),
            pltpu.VMEM((_CHUNK,), jnp.float32),
            pltpu.SemaphoreType.DMA,
        ],
    )
    return run(stu_id, input_exercise, table)


# speedup vs baseline: 2.1671x; 2.1671x over previous
import jax
import jax.numpy as jnp
from jax import lax
from jax.experimental import pallas as pl
from jax.experimental.pallas import tpu as pltpu
from jax.experimental.pallas import tpu_sc as plsc

_BATCH = 16384
_NC = 2
_NS = 16
_L = 16
_NW = _NC * _NS
_CHUNK = _BATCH // _NW

_VALUE_RANGE = 8.0
_A_RANGE = 3.0


def _sigmoid(x):
    return 1.0 / (1.0 + jnp.exp(-x))


def _body(stu_hbm, exer_hbm, theta_hbm, a_hbm, b_hbm, c_hbm, out_hbm,
          idx_s, idx_e, th_v, a_v, b_v, c_v, out_v, sem):
    wid = lax.axis_index("s") * _NC + lax.axis_index("c")
    base = wid * _CHUNK
    ci1 = pltpu.async_copy(stu_hbm.at[pl.ds(base, _CHUNK)], idx_s, sem)
    ci2 = pltpu.async_copy(exer_hbm.at[pl.ds(base, _CHUNK)], idx_e, sem)
    ci1.wait()
    cp1 = pltpu.async_copy(theta_hbm.at[idx_s], th_v, sem)
    ci2.wait()
    cp2 = pltpu.async_copy(a_hbm.at[idx_e], a_v, sem)
    cp3 = pltpu.async_copy(b_hbm.at[idx_e], b_v, sem)
    cp4 = pltpu.async_copy(c_hbm.at[idx_e], c_v, sem)
    cp1.wait()
    cp2.wait()
    cp3.wait()
    cp4.wait()
    for i in range(_CHUNK // _L):
        sl = pl.ds(i * _L, _L)
        th = _VALUE_RANGE * (_sigmoid(th_v[sl]) - 0.5)
        bb = _VALUE_RANGE * (_sigmoid(b_v[sl]) - 0.5)
        aa = _A_RANGE * _sigmoid(a_v[sl])
        cc = _sigmoid(c_v[sl])
        out_v[sl] = cc + (1.0 - cc) / (1.0 + jnp.exp(-1.702 * aa * (th - bb)))
    pltpu.sync_copy(out_v, out_hbm.at[pl.ds(base, _CHUNK)])


def kernel(stu_id, input_exercise, theta_w, a_w, b_w, c_w):
    mesh = plsc.VectorSubcoreMesh(
        core_axis_name="c", subcore_axis_name="s",
        num_cores=_NC, num_subcores=_NS)
    run = pl.kernel(
        _body,
        out_type=jax.ShapeDtypeStruct((_BATCH,), jnp.float32),
        mesh=mesh,
        compiler_params=pltpu.CompilerParams(skip_device_barrier=True),
        scratch_types=[
            pltpu.VMEM((_CHUNK,), jnp.int32),
            pltpu.VMEM((_CHUNK,), jnp.int32),
            pltpu.VMEM((_CHUNK,), jnp.float32),
            pltpu.VMEM((_CHUNK,), jnp.float32),
            pltpu.VMEM((_CHUNK,), jnp.float32),
            pltpu.VMEM((_CHUNK,), jnp.float32),
            pltpu.VMEM((_CHUNK,), jnp.float32),
            pltpu.SemaphoreType.DMA,
        ],
    )
    return run(stu_id, input_exercise,
               theta_w.reshape(-1), a_w.reshape(-1),
               b_w.reshape(-1), c_w.reshape(-1))
